# baseline (device time: 87938 ns/iter reference)
import jax
import jax.numpy as jnp
from jax import lax
from jax.experimental import pallas as pl
from jax.experimental.pallas import tpu as pltpu

N_DEV = 4
N_GLOBAL = 8192
EPS = 1e-5
ROWS_C = 48
BLK = 4
N_BLOCKS = ROWS_C // BLK
LANES = 128


def _fused_body(x_ref, gamma_ref, out_ref, psum_ref, comm_ref,
                scale_ref, send_sems, recv_sems):
    b = pl.program_id(0)
    me = lax.axis_index("i")

    @pl.when(b < N_BLOCKS)
    def _phase_a():
        x = x_ref[...]
        x2 = x * x
        f = x2[:, :, 0:LANES]
        for k in range(1, x2.shape[2] // LANES):
            f = f + x2[:, :, LANES * k:LANES * (k + 1)]
        psum_ref[pl.ds(BLK * b, BLK), :] = jnp.sum(f, axis=2)

    @pl.when(b == N_BLOCKS - 1)
    def _allreduce():
        barrier = pltpu.get_barrier_semaphore()
        for k in range(1, N_DEV):
            peer = (me + k) % N_DEV
            pl.semaphore_signal(
                barrier, inc=1,
                device_id=(peer,), device_id_type=pl.DeviceIdType.MESH,
            )
        pl.semaphore_wait(barrier, N_DEV - 1)

        comm_ref[me] = psum_ref[...]

        sends = []
        for k in range(1, N_DEV):
            peer = (me + k) % N_DEV
            rdma = pltpu.make_async_remote_copy(
                src_ref=comm_ref.at[me],
                dst_ref=comm_ref.at[me],
                send_sem=send_sems.at[k - 1],
                recv_sem=recv_sems.at[me],
                device_id=(peer,),
                device_id_type=pl.DeviceIdType.MESH,
            )
            rdma.start()
            sends.append(rdma)

        for k in range(1, N_DEV):
            peer = (me + k) % N_DEV
            recv = pltpu.make_async_remote_copy(
                src_ref=comm_ref.at[peer],
                dst_ref=comm_ref.at[peer],
                send_sem=send_sems.at[k - 1],
                recv_sem=recv_sems.at[peer],
                device_id=(peer,),
                device_id_type=pl.DeviceIdType.MESH,
            )
            recv.wait_recv()
        for s in sends:
            s.wait_send()

        total = comm_ref[0] + comm_ref[1] + comm_ref[2] + comm_ref[3]
        scale_ref[...] = lax.rsqrt(total * (1.0 / N_GLOBAL) + EPS)

    @pl.when(b >= N_BLOCKS)
    def _phase_b():
        j = b - N_BLOCKS
        g = gamma_ref[...][None, :, :]
        s = scale_ref[pl.ds(BLK * j, BLK), :][:, :, None]
        out_ref[...] = x_ref[...] * g * s


def kernel(x, gamma):
    m, n_local = x.shape
    x3 = x.reshape(ROWS_C, 128, n_local)
    gamma2 = gamma.reshape(1, n_local)

    out3 = pl.pallas_call(
        _fused_body,
        grid=(2 * N_BLOCKS,),
        in_specs=[
            pl.BlockSpec((BLK, 128, n_local), lambda b: (b % N_BLOCKS, 0, 0)),
            pl.BlockSpec((1, n_local), lambda b: (0, 0)),
        ],
        out_specs=pl.BlockSpec(
            (BLK, 128, n_local),
            lambda b: (jnp.maximum(b - N_BLOCKS, 0), 0, 0),
        ),
        out_shape=jax.ShapeDtypeStruct((ROWS_C, 128, n_local), jnp.float32),
        scratch_shapes=[
            pltpu.VMEM((ROWS_C, 128), jnp.float32),
            pltpu.VMEM((N_DEV, ROWS_C, 128), jnp.float32),
            pltpu.VMEM((ROWS_C, 128), jnp.float32),
            pltpu.SemaphoreType.DMA((N_DEV - 1,)),
            pltpu.SemaphoreType.DMA((N_DEV,)),
        ],
        compiler_params=pltpu.CompilerParams(
            collective_id=0,
            vmem_limit_bytes=100 * 1024 * 1024,
        ),
    )(x3, gamma2)

    return out3.reshape(m, n_local)


# device time: 73216 ns/iter; 1.2011x vs baseline; 1.2011x over previous
import jax
import jax.numpy as jnp
from jax import lax
from jax.experimental import pallas as pl
from jax.experimental.pallas import tpu as pltpu

N_DEV = 4
N_GLOBAL = 8192
EPS = 1e-5
ROWS_C = 48
BLK = 4
N_BLOCKS = ROWS_C // BLK
LANES = 128


def _in_copy(x_ref, stash_ref, in_sems, b):
    sl = pl.ds(BLK * b, BLK)
    return pltpu.make_async_copy(x_ref.at[sl], stash_ref.at[sl], in_sems.at[b])


def _out_copy(obuf_ref, out_ref, out_sems, b):
    sl = pl.ds(BLK * b, BLK)
    return pltpu.make_async_copy(obuf_ref.at[b % 2], out_ref.at[sl], out_sems.at[b])


def _body(x_ref, gamma_ref, out_ref, stash_ref, obuf_ref, psum_ref, comm_ref,
          scale_ref, in_sems, out_sems, send_sems, recv_sems):
    me = lax.axis_index("i")

    for b in range(N_BLOCKS):
        _in_copy(x_ref, stash_ref, in_sems, b).start()

    for b in range(N_BLOCKS):
        _in_copy(x_ref, stash_ref, in_sems, b).wait()
        xx = stash_ref[pl.ds(BLK * b, BLK)]
        x2 = xx * xx
        f = x2[:, :, 0:LANES]
        for k in range(1, x2.shape[2] // LANES):
            f = f + x2[:, :, LANES * k:LANES * (k + 1)]
        psum_ref[pl.ds(BLK * b, BLK), :] = jnp.sum(f, axis=2)

    barrier = pltpu.get_barrier_semaphore()
    for k in range(1, N_DEV):
        peer = (me + k) % N_DEV
        pl.semaphore_signal(
            barrier, inc=1,
            device_id=(peer,), device_id_type=pl.DeviceIdType.MESH,
        )
    pl.semaphore_wait(barrier, N_DEV - 1)

    comm_ref[me] = psum_ref[...]

    sends = []
    for k in range(1, N_DEV):
        peer = (me + k) % N_DEV
        rdma = pltpu.make_async_remote_copy(
            src_ref=comm_ref.at[me],
            dst_ref=comm_ref.at[me],
            send_sem=send_sems.at[k - 1],
            recv_sem=recv_sems.at[me],
            device_id=(peer,),
            device_id_type=pl.DeviceIdType.MESH,
        )
        rdma.start()
        sends.append(rdma)

    for k in range(1, N_DEV):
        peer = (me + k) % N_DEV
        recv = pltpu.make_async_remote_copy(
            src_ref=comm_ref.at[peer],
            dst_ref=comm_ref.at[peer],
            send_sem=send_sems.at[k - 1],
            recv_sem=recv_sems.at[peer],
            device_id=(peer,),
            device_id_type=pl.DeviceIdType.MESH,
        )
        recv.wait_recv()
    for s in sends:
        s.wait_send()

    total = comm_ref[0] + comm_ref[1] + comm_ref[2] + comm_ref[3]
    scale_ref[...] = lax.rsqrt(total * (1.0 / N_GLOBAL) + EPS)

    g = gamma_ref[...][None, :, :]
    for b in range(N_BLOCKS):
        if b >= 2:
            _out_copy(obuf_ref, out_ref, out_sems, b - 2).wait()
        s = scale_ref[pl.ds(BLK * b, BLK), :][:, :, None]
        obuf_ref[b % 2] = stash_ref[pl.ds(BLK * b, BLK)] * g * s
        _out_copy(obuf_ref, out_ref, out_sems, b).start()

    for b in range(N_BLOCKS - 2, N_BLOCKS):
        _out_copy(obuf_ref, out_ref, out_sems, b).wait()


def kernel(x, gamma):
    m, n_local = x.shape
    x3 = x.reshape(ROWS_C, 128, n_local)
    gamma2 = gamma.reshape(1, n_local)

    out3 = pl.pallas_call(
        _body,
        in_specs=[
            pl.BlockSpec(memory_space=pltpu.MemorySpace.HBM),
            pl.BlockSpec(memory_space=pltpu.VMEM),
        ],
        out_specs=pl.BlockSpec(memory_space=pltpu.MemorySpace.HBM),
        out_shape=jax.ShapeDtypeStruct((ROWS_C, 128, n_local), jnp.float32),
        scratch_shapes=[
            pltpu.VMEM((ROWS_C, 128, n_local), jnp.float32),
            pltpu.VMEM((2, BLK, 128, n_local), jnp.float32),
            pltpu.VMEM((ROWS_C, 128), jnp.float32),
            pltpu.VMEM((N_DEV, ROWS_C, 128), jnp.float32),
            pltpu.VMEM((ROWS_C, 128), jnp.float32),
            pltpu.SemaphoreType.DMA((N_BLOCKS,)),
            pltpu.SemaphoreType.DMA((N_BLOCKS,)),
            pltpu.SemaphoreType.DMA((N_DEV - 1,)),
            pltpu.SemaphoreType.DMA((N_DEV,)),
        ],
        compiler_params=pltpu.CompilerParams(
            collective_id=0,
            vmem_limit_bytes=100 * 1024 * 1024,
        ),
    )(x3, gamma2)

    return out3.reshape(m, n_local)
